# baseline (speedup 1.0000x reference)
import functools

import jax
import jax.numpy as jnp
from jax import lax
from jax.experimental import pallas as pl
from jax.experimental.pallas import tpu as pltpu
from jax.experimental.pallas import tpu_sc as plsc

EMBED = 64
T = 2
NC, NS = 2, 16
NW = NC * NS
CHUNK = 128
NREG = EMBED // 16


def _make_sc_kernel(B, S):
    ROWS = B * S
    assert ROWS % NW == 0
    RPW = ROWS // NW
    assert RPW % CHUNK == 0 and RPW % S == 0
    NCH = RPW // CHUNK

    mesh = plsc.VectorSubcoreMesh(
        core_axis_name="c", subcore_axis_name="s",
        num_cores=NC, num_subcores=NS)

    @functools.partial(
        pl.kernel,
        out_type=jax.ShapeDtypeStruct((T, ROWS, EMBED), jnp.float32),
        mesh=mesh,
        compiler_params=pltpu.CompilerParams(use_tc_tiling_on_sc=False),
        scratch_types=[
            pltpu.VMEM((NCH, CHUNK), jnp.int32),
            pltpu.VMEM((S, EMBED), jnp.float32),
            pltpu.VMEM((EMBED,), jnp.float32),
            pltpu.VMEM((EMBED,), jnp.float32),
            pltpu.VMEM((CHUNK, EMBED), jnp.float32),
            pltpu.VMEM((CHUNK, EMBED), jnp.float32),
            pltpu.VMEM((16,), jnp.float32),
            pltpu.SemaphoreType.DMA,
        ],
    )
    def sc_kernel(x_hbm, tab_hbm, gam_hbm, bet_hbm, poe_hbm, out_hbm,
                  idx_v, poe_v, gam_v, bet_v, g_v, o_v, var_v, sem):
        wid = lax.axis_index("s") * NC + lax.axis_index("c")
        base_row = wid * RPW
        pltpu.sync_copy(x_hbm.at[wid], idx_v)
        pltpu.sync_copy(poe_hbm, poe_v)
        pltpu.sync_copy(gam_hbm, gam_v)
        pltpu.sync_copy(bet_hbm, bet_v)
        gam = [gam_v[pl.ds(16 * j, 16)] for j in range(NREG)]
        bet = [bet_v[pl.ds(16 * j, 16)] for j in range(NREG)]
        lanes = lax.iota(jnp.int32, 16)
        perms = [lanes ^ k for k in (1, 2, 4, 8)]

        def allsum(v):
            for p in perms:
                v = v + v.at[p].get(mode="promise_in_bounds")
            return v

        def chunk_body(c, s0):
            pltpu.async_copy(tab_hbm.at[idx_v.at[c]], g_v, sem).wait()

            def row_body(r, s):
                x = [g_v[r, pl.ds(16 * j, 16)] + poe_v[s, pl.ds(16 * j, 16)]
                     for j in range(NREG)]
                tot = allsum((x[0] + x[1]) + (x[2] + x[3]))
                tot2 = allsum((x[0] * x[0] + x[1] * x[1])
                              + (x[2] * x[2] + x[3] * x[3]))
                mean = tot * (1.0 / EMBED)
                v = tot2 * (1.0 / EMBED) - mean * mean + 1e-5
                ib = lax.bitcast_convert_type(v[0], jnp.int32)
                ib = jnp.int32(0x5F3759DF) - (ib >> 1)
                y0 = lax.bitcast_convert_type(ib, jnp.float32)
                y = jnp.full((16,), y0, jnp.float32)
                for _ in range(3):
                    y = y * (1.5 - 0.5 * v * y * y)
                for j in range(NREG):
                    a = gam[j] * y
                    b = bet[j] - mean * a
                    o_v[r, pl.ds(16 * j, 16)] = x[j] * a + b
                s = s + 1
                return jnp.where(s == S, 0, s)

            s1 = lax.fori_loop(0, CHUNK, row_body, s0)
            row0 = base_row + c * CHUNK
            pltpu.sync_copy(o_v, out_hbm.at[0, pl.ds(row0, CHUNK), :])
            pltpu.sync_copy(o_v, out_hbm.at[1, pl.ds(row0, CHUNK), :])
            return s1

        lax.fori_loop(0, NCH, chunk_body, jnp.int32(0))

    return sc_kernel


def kernel(x, emb_table, ln_gamma, ln_beta, poe):
    B, S = x.shape
    ROWS = B * S
    x_r = x.astype(jnp.int32).reshape(NW, ROWS // (NW * CHUNK), CHUNK)
    poe_s = poe[:S]
    out = _make_sc_kernel(B, S)(x_r, emb_table, ln_gamma, ln_beta, poe_s)
    return out.reshape(T, B, S, EMBED)


# speedup vs baseline: 1.0591x; 1.0591x over previous
import functools

import jax
import jax.numpy as jnp
from jax import lax
from jax.experimental import pallas as pl
from jax.experimental.pallas import tpu as pltpu
from jax.experimental.pallas import tpu_sc as plsc

EMBED = 64
T = 2
NC, NS = 2, 16
NW = NC * NS
HALF = 100
NREG = EMBED // 16
UNROLL = 4


def _make_sc_kernel(B, S):
    ROWS = B * S
    assert ROWS % NW == 0
    RPW = ROWS // NW
    assert RPW % S == 0 and S == 2 * HALF
    NSEQ = RPW // S
    assert NSEQ % 2 == 0

    mesh = plsc.VectorSubcoreMesh(
        core_axis_name="c", subcore_axis_name="s",
        num_cores=NC, num_subcores=NS)

    @functools.partial(
        pl.kernel,
        out_type=jax.ShapeDtypeStruct((T, ROWS, EMBED), jnp.float32),
        mesh=mesh,
        compiler_params=pltpu.CompilerParams(use_tc_tiling_on_sc=False),
        scratch_types=[
            pltpu.VMEM((NSEQ, 2, HALF), jnp.int32),
            pltpu.VMEM((S, EMBED), jnp.float32),
            pltpu.VMEM((EMBED,), jnp.float32),
            pltpu.VMEM((EMBED,), jnp.float32),
            pltpu.VMEM((S, EMBED), jnp.float32),
            pltpu.VMEM((S, EMBED), jnp.float32),
            pltpu.VMEM((S, EMBED), jnp.float32),
            pltpu.VMEM((S, EMBED), jnp.float32),
            pltpu.SemaphoreType.DMA,
            pltpu.SemaphoreType.DMA,
        ],
    )
    def sc_kernel(x_hbm, tab_hbm, gam_hbm, bet_hbm, poe_hbm, out_hbm,
                  idx_v, poe_v, gam_v, bet_v, g0, g1, o0, o1, sem_g, sem_o):
        wid = lax.axis_index("s") * NC + lax.axis_index("c")
        base_row = wid * RPW
        pltpu.sync_copy(x_hbm.at[wid], idx_v)
        pltpu.sync_copy(poe_hbm, poe_v)
        pltpu.sync_copy(gam_hbm, gam_v)
        pltpu.sync_copy(bet_hbm, bet_v)
        gam = [gam_v[pl.ds(16 * j, 16)] for j in range(NREG)]
        bet = [bet_v[pl.ds(16 * j, 16)] for j in range(NREG)]
        lanes = lax.iota(jnp.int32, 16)
        perms = [lanes ^ k for k in (1, 2, 4, 8)]
        gbuf = (g0, g1)
        obuf = (o0, o1)

        def allsum(v):
            for p in perms:
                v = v + v.at[p].get(mode="promise_in_bounds")
            return v

        def fire_gather(c, g):
            for h in range(2):
                pltpu.make_async_copy(
                    tab_hbm.at[idx_v.at[c, h]],
                    g.at[pl.ds(h * HALF, HALF), :], sem_g).start()

        def wait_gather(g):
            for h in range(2):
                pltpu.make_async_copy(
                    tab_hbm.at[idx_v.at[0, 0]],
                    g.at[pl.ds(h * HALF, HALF), :], sem_g).wait()

        def fire_out(c, o):
            row0 = base_row + c * S
            for t in range(T):
                pltpu.make_async_copy(
                    o, out_hbm.at[t, pl.ds(row0, S), :], sem_o).start()

        def wait_out(o):
            for t in range(T):
                pltpu.make_async_copy(
                    o, out_hbm.at[t, pl.ds(base_row, S), :], sem_o).wait()

        def compute(g, o):
            def row_body(r, carry):
                x = [g[r, pl.ds(16 * j, 16)] + poe_v[r, pl.ds(16 * j, 16)]
                     for j in range(NREG)]
                tot = allsum((x[0] + x[1]) + (x[2] + x[3]))
                tot2 = allsum((x[0] * x[0] + x[1] * x[1])
                              + (x[2] * x[2] + x[3] * x[3]))
                mean = tot * (1.0 / EMBED)
                v = tot2 * (1.0 / EMBED) - mean * mean + 1e-5
                ib = lax.bitcast_convert_type(v[0], jnp.int32)
                ib = jnp.int32(0x5F3759DF) - (ib >> 1)
                y = jnp.full((16,), lax.bitcast_convert_type(ib, jnp.float32),
                             jnp.float32)
                for _ in range(3):
                    y = y * (1.5 - 0.5 * v * y * y)
                for j in range(NREG):
                    a = gam[j] * y
                    b = bet[j] - mean * a
                    o[r, pl.ds(16 * j, 16)] = x[j] * a + b
                return carry

            lax.fori_loop(0, S, row_body, 0, unroll=UNROLL)

        fire_gather(0, g0)
        fire_gather(1, g1)

        def outer(cc, carry):
            for b in range(2):
                c = 2 * cc + b
                wait_gather(gbuf[b])

                @pl.when(cc >= 1)
                def _():
                    wait_out(obuf[b])

                compute(gbuf[b], obuf[b])
                fire_out(c, obuf[b])

                @pl.when(cc < NSEQ // 2 - 1)
                def _():
                    fire_gather(c + 2, gbuf[b])

            return carry

        lax.fori_loop(0, NSEQ // 2, outer, 0)
        wait_out(o0)
        wait_out(o1)

    return sc_kernel


def kernel(x, emb_table, ln_gamma, ln_beta, poe):
    B, S = x.shape
    ROWS = B * S
    x_r = x.astype(jnp.int32).reshape(NW, ROWS // (NW * S), 2, HALF)
    poe_s = poe[:S]
    out = _make_sc_kernel(B, S)(x_r, emb_table, ln_gamma, ln_beta, poe_s)
    return out.reshape(T, B, S, EMBED)
